# baseline (speedup 1.0000x reference)
import functools

import jax
import jax.numpy as jnp
from jax import lax
from jax.experimental import pallas as pl
from jax.experimental.pallas import tpu as pltpu
from jax.experimental.pallas import tpu_sc as plsc

B, N, M, F = 64, 128, 128, 16
INF = 1000000.0
LN2 = 0.6931471805599453
LOG2E = 1.4426950408889634
NCHUNK = 9
W = NCHUNK * 16
PADN = 18432
SQRT2 = 1.4142135

E2 = (1.0000000754953546, 0.6931210339915476, 0.2402210735581854,
      0.05592203564725779, 0.009676037098297214)
L2 = (-1.5931662034414242e-06, 1.4427138430920823, -0.721039027838375,
      0.4793922956076517, -0.36882168336927695, 0.3216487808117249,
      -0.20098562628855351)


def _dist_body(a_ref, b_ref, o_ref):
    a = a_ref[...]
    b = b_ref[...]
    ab = lax.dot_general(a, b, (((2,), (2,)), ((0,), (0,))),
                         preferred_element_type=jnp.float32)
    na = jnp.sum(a * a, axis=-1)
    nb = jnp.sum(b * b, axis=-1)
    d = (na[:, :, None] + nb[:, None, :] - 2.0 * ab) * LOG2E
    o_ref[:, 128:128 + N * M] = d.reshape(B, N * M)
    o_ref[:, 0:128] = jnp.full((B, 128), INF, jnp.float32)
    o_ref[:, 128 + N * M:] = jnp.full((B, PADN - 128 - N * M), INF,
                                      jnp.float32)


def _dist(a, b):
    return pl.pallas_call(
        _dist_body,
        out_shape=jax.ShapeDtypeStruct((B, PADN), jnp.float32),
    )(a, b)


def _exp2(x):
    x = jnp.maximum(x, -125.0)
    k = (x - 0.5).astype(jnp.int32)
    f = x - k.astype(jnp.float32)
    p = E2[4]
    for c in (E2[3], E2[2], E2[1], E2[0]):
        p = p * f + c
    scale = plsc.bitcast(lax.shift_left(k + 127, 23), jnp.float32)
    return p * scale


def _log2(z):
    zi = plsc.bitcast(z, jnp.int32)
    e = lax.shift_right_logical(zi, 23) - 127
    mant = plsc.bitcast((zi & 0x007FFFFF) | 0x3F800000, jnp.float32)
    big = mant > SQRT2
    mant = jnp.where(big, 0.5 * mant, mant)
    e = e + jnp.where(big, 1, 0)
    t = mant - 1.0
    p = L2[6]
    for c in (L2[5], L2[4], L2[3], L2[2], L2[1], L2[0]):
        p = p * t + c
    return e.astype(jnp.float32) + p


@functools.partial(
    pl.kernel,
    out_type=jax.ShapeDtypeStruct((B, 16), jnp.float32),
    mesh=plsc.VectorSubcoreMesh(core_axis_name="c", subcore_axis_name="s"),
    compiler_params=pltpu.CompilerParams(needs_layout_passes=False),
    scratch_types=[
        pltpu.VMEM((PADN,), jnp.float32),
        pltpu.VMEM((PADN,), jnp.float32),
        pltpu.VMEM((W,), jnp.float32),
        pltpu.VMEM((W,), jnp.float32),
        pltpu.VMEM((W,), jnp.float32),
        pltpu.VMEM((W,), jnp.float32),
        pltpu.VMEM((W,), jnp.float32),
        pltpu.VMEM((W,), jnp.float32),
        pltpu.VMEM((16,), jnp.float32),
    ],
)
def _sc_dp(d_hbm, out_hbm, dva, dvb, a0, a1, a2, b0, b1, b2, o_v):
    nc = plsc.get_sparse_core_info().num_cores
    wid = lax.axis_index("s") * nc + lax.axis_index("c")
    iota = lax.iota(jnp.int32, 16)
    inf_v = jnp.full((16,), INF, jnp.float32)
    pair = wid * 2
    pltpu.sync_copy(d_hbm.at[pair], dva)
    pltpu.sync_copy(d_hbm.at[pair + 1], dvb)

    for c in range(NCHUNK):
        sl = pl.ds(c * 16, 16)
        v0 = jnp.where(iota == 0, 0.0, INF) if c == 0 else inf_v
        a0[sl] = v0
        b0[sl] = v0
        a1[sl] = inf_v
        b1[sl] = inf_v
        a2[sl] = inf_v
        b2[sl] = inf_v

    def chunk(c, t, cur, p1, p2, dv):
        ivec = c * 16 + iota
        ish = jnp.maximum(ivec - 1, 0)
        r_up = p1[pl.ds(c * 16, 16)]
        r_left = plsc.load_gather(p1, [ish])
        r_dd = plsc.load_gather(p2, [ish])
        d = plsc.load_gather(dv, [127 * ivec + (t - 1)])
        m3 = jnp.minimum(jnp.minimum(r_left, r_up), r_dd)
        z = _exp2(m3 - r_left) + _exp2(m3 - r_up) + _exp2(m3 - r_dd)
        cur[pl.ds(c * 16, 16)] = d + (m3 - _log2(z))

    def diag(t, ca, p1a, p2a, cb, p1b, p2b):
        for c in range(NCHUNK):
            chunk(c, t, ca, p1a, p2a, dva)
            chunk(c, t, cb, p1b, p2b, dvb)

    def body(k, carry):
        t = 3 * k + 2
        diag(t, a2, a1, a0, b2, b1, b0)
        diag(t + 1, a0, a2, a1, b0, b2, b1)
        diag(t + 2, a1, a0, a2, b1, b0, b2)
        return carry

    lax.fori_loop(0, (N + M - 1) // 3, body, 0)
    o_v[...] = a1[pl.ds(128, 16)] * LN2
    pltpu.sync_copy(o_v, out_hbm.at[pair])
    o_v[...] = b1[pl.ds(128, 16)] * LN2
    pltpu.sync_copy(o_v, out_hbm.at[pair + 1])


def kernel(seq_a, seq_b):
    d = _dist(seq_a, seq_b)
    out = _sc_dp(d)
    return out[:, 0:1]


# speedup vs baseline: 1.2982x; 1.2982x over previous
import functools

import jax
import jax.numpy as jnp
from jax import lax
from jax.experimental import pallas as pl
from jax.experimental.pallas import tpu as pltpu
from jax.experimental.pallas import tpu_sc as plsc

B, N, M, F = 64, 128, 128, 16
INF = 1000000.0
LN2 = 0.6931471805599453
LOG2E = 1.4426950408889634
NCHUNK = 9
W = NCHUNK * 16
PADN = 18432
SQRT2 = 1.4142135

E2 = (1.0000000754953546, 0.6931210339915476, 0.2402210735581854,
      0.05592203564725779, 0.009676037098297214)
L2 = (-1.5931662034414242e-06, 1.4427138430920823, -0.721039027838375,
      0.4793922956076517, -0.36882168336927695, 0.3216487808117249,
      -0.20098562628855351)


def _dist_body(a_ref, b_ref, o_ref):
    a = a_ref[...]
    b = b_ref[...]
    ab = lax.dot_general(a, b, (((2,), (2,)), ((0,), (0,))),
                         preferred_element_type=jnp.float32)
    na = jnp.sum(a * a, axis=-1)
    nb = jnp.sum(b * b, axis=-1)
    d = (na[:, :, None] + nb[:, None, :] - 2.0 * ab) * LOG2E
    o_ref[:, 128:128 + N * M] = d.reshape(B, N * M)
    o_ref[:, 0:128] = jnp.full((B, 128), INF, jnp.float32)
    o_ref[:, 128 + N * M:] = jnp.full((B, PADN - 128 - N * M), INF,
                                      jnp.float32)


def _dist(a, b):
    return pl.pallas_call(
        _dist_body,
        out_shape=jax.ShapeDtypeStruct((B, PADN), jnp.float32),
    )(a, b)


def _exp2(x):
    return jnp.exp(LN2 * x)


def _log2(z):
    zi = plsc.bitcast(z, jnp.int32)
    e = lax.shift_right_logical(zi, 23) - 127
    mant = plsc.bitcast((zi & 0x007FFFFF) | 0x3F800000, jnp.float32)
    big = mant > SQRT2
    mant = jnp.where(big, 0.5 * mant, mant)
    e = e + jnp.where(big, 1, 0)
    t = mant - 1.0
    p = L2[6]
    for c in (L2[5], L2[4], L2[3], L2[2], L2[1], L2[0]):
        p = p * t + c
    return e.astype(jnp.float32) + p


@functools.partial(
    pl.kernel,
    out_type=jax.ShapeDtypeStruct((B, 16), jnp.float32),
    mesh=plsc.VectorSubcoreMesh(core_axis_name="c", subcore_axis_name="s"),
    compiler_params=pltpu.CompilerParams(needs_layout_passes=False),
    scratch_types=[
        pltpu.VMEM((PADN,), jnp.float32),
        pltpu.VMEM((PADN,), jnp.float32),
        pltpu.VMEM((W,), jnp.float32),
        pltpu.VMEM((W,), jnp.float32),
        pltpu.VMEM((W,), jnp.float32),
        pltpu.VMEM((W,), jnp.float32),
        pltpu.VMEM((W,), jnp.float32),
        pltpu.VMEM((W,), jnp.float32),
        pltpu.VMEM((16,), jnp.float32),
    ],
)
def _sc_dp(d_hbm, out_hbm, dva, dvb, a0, a1, a2, b0, b1, b2, o_v):
    nc = plsc.get_sparse_core_info().num_cores
    wid = lax.axis_index("s") * nc + lax.axis_index("c")
    iota = lax.iota(jnp.int32, 16)
    inf_v = jnp.full((16,), INF, jnp.float32)
    pair = wid * 2
    pltpu.sync_copy(d_hbm.at[pair], dva)
    pltpu.sync_copy(d_hbm.at[pair + 1], dvb)

    for c in range(NCHUNK):
        sl = pl.ds(c * 16, 16)
        v0 = jnp.where(iota == 0, 0.0, INF) if c == 0 else inf_v
        a0[sl] = v0
        b0[sl] = v0
        a1[sl] = inf_v
        b1[sl] = inf_v
        a2[sl] = inf_v
        b2[sl] = inf_v

    def chunk(c, t, cur, p1, p2, dv):
        ivec = c * 16 + iota
        ish = jnp.maximum(ivec - 1, 0)
        r_up = p1[pl.ds(c * 16, 16)]
        r_left = plsc.load_gather(p1, [ish])
        r_dd = plsc.load_gather(p2, [ish])
        d = plsc.load_gather(dv, [127 * ivec + (t - 1)])
        m3 = jnp.minimum(jnp.minimum(r_left, r_up), r_dd)
        z = _exp2(m3 - r_left) + _exp2(m3 - r_up) + _exp2(m3 - r_dd)
        cur[pl.ds(c * 16, 16)] = d + (m3 - _log2(z))

    def diag(t, ca, p1a, p2a, cb, p1b, p2b):
        for c in range(NCHUNK):
            chunk(c, t, ca, p1a, p2a, dva)
            chunk(c, t, cb, p1b, p2b, dvb)

    def body(k, carry):
        t = 3 * k + 2
        diag(t, a2, a1, a0, b2, b1, b0)
        diag(t + 1, a0, a2, a1, b0, b2, b1)
        diag(t + 2, a1, a0, a2, b1, b0, b2)
        return carry

    lax.fori_loop(0, (N + M - 1) // 3, body, 0)
    o_v[...] = a1[pl.ds(128, 16)] * LN2
    pltpu.sync_copy(o_v, out_hbm.at[pair])
    o_v[...] = b1[pl.ds(128, 16)] * LN2
    pltpu.sync_copy(o_v, out_hbm.at[pair + 1])


def kernel(seq_a, seq_b):
    d = _dist(seq_a, seq_b)
    out = _sc_dp(d)
    return out[:, 0:1]


# speedup vs baseline: 5.5830x; 4.3005x over previous
import functools

import jax
import jax.numpy as jnp
from jax import lax
from jax.experimental import pallas as pl
from jax.experimental.pallas import tpu as pltpu
from jax.experimental.pallas import tpu_sc as plsc

B, N, M, F = 64, 128, 128, 16
INF = 1000000.0
LN2 = 0.6931471805599453
NCHUNK = 9
W = NCHUNK * 16
PADN = 18432
SQRT2 = 1.4142135

L2 = (-1.5931662034414242e-06, 1.4427138430920823, -0.721039027838375,
      0.4793922956076517, -0.36882168336927695, 0.3216487808117249,
      -0.20098562628855351)


def _dist_body(a_ref, b_ref, o_ref):
    a = a_ref[...]
    b = b_ref[...]
    ab = lax.dot_general(a, b, (((2,), (2,)), ((0,), (0,))),
                         preferred_element_type=jnp.float32)
    na = jnp.sum(a * a, axis=-1)
    nb = jnp.sum(b * b, axis=-1)
    d = na[:, :, None] + nb[:, None, :] - 2.0 * ab
    o_ref[:, 128:128 + N * M] = d.reshape(B, N * M)
    o_ref[:, 0:128] = jnp.full((B, 128), INF, jnp.float32)
    o_ref[:, 128 + N * M:] = jnp.full((B, PADN - 128 - N * M), INF,
                                      jnp.float32)


def _dist(a, b):
    return pl.pallas_call(
        _dist_body,
        out_shape=jax.ShapeDtypeStruct((B, PADN), jnp.float32),
    )(a, b)


@functools.partial(
    pl.kernel,
    out_type=jax.ShapeDtypeStruct((B, 16), jnp.float32),
    mesh=plsc.VectorSubcoreMesh(core_axis_name="c", subcore_axis_name="s"),
    compiler_params=pltpu.CompilerParams(needs_layout_passes=False),
    scratch_types=[
        pltpu.VMEM((PADN,), jnp.float32),
        pltpu.VMEM((PADN,), jnp.float32),
        pltpu.VMEM((W,), jnp.float32),
        pltpu.VMEM((W,), jnp.float32),
        pltpu.VMEM((W,), jnp.float32),
        pltpu.VMEM((W,), jnp.float32),
        pltpu.VMEM((W,), jnp.float32),
        pltpu.VMEM((W,), jnp.float32),
        pltpu.VMEM((16,), jnp.float32),
    ],
)
def _sc_dp(d_hbm, out_hbm, dva, dvb, a0, a1, a2, b0, b1, b2, o_v):
    nc = plsc.get_sparse_core_info().num_cores
    wid = lax.axis_index("s") * nc + lax.axis_index("c")
    iota = lax.iota(jnp.int32, 16)
    inf_v = jnp.full((16,), INF, jnp.float32)
    pair = wid * 2
    pltpu.sync_copy(d_hbm.at[pair], dva)
    pltpu.sync_copy(d_hbm.at[pair + 1], dvb)

    for c in range(NCHUNK):
        sl = pl.ds(c * 16, 16)
        v0 = jnp.where(iota == 0, 0.0, INF) if c == 0 else inf_v
        a0[sl] = v0
        b0[sl] = v0
        a1[sl] = inf_v
        b1[sl] = inf_v
        a2[sl] = inf_v
        b2[sl] = inf_v

    C = NCHUNK
    R = range(C)

    def diag(t, cur, p1, p2, dv):
        ish0 = jnp.maximum(iota - 1, 0)
        r_up, r_lf, r_dd, dval = [], [], [], []
        for c in R:
            ish = ish0 if c == 0 else iota + (16 * c - 1)
            r_up.append(p1[pl.ds(c * 16, 16)])
            r_lf.append(plsc.load_gather(p1, [ish]))
            r_dd.append(plsc.load_gather(p2, [ish]))
            dval.append(plsc.load_gather(dv, [iota * 127 + (2032 * c + t - 1)]))
        m3 = [jnp.minimum(jnp.minimum(r_lf[c], r_up[c]), r_dd[c]) for c in R]
        e1 = [jnp.exp(m3[c] - r_lf[c]) for c in R]
        e2 = [jnp.exp(m3[c] - r_up[c]) for c in R]
        e3 = [jnp.exp(m3[c] - r_dd[c]) for c in R]
        z = [(e1[c] + e2[c]) + e3[c] for c in R]
        zi = [plsc.bitcast(z[c], jnp.int32) for c in R]
        ex = [lax.shift_right_logical(zi[c], 23) - 127 for c in R]
        mant = [plsc.bitcast((zi[c] & 0x007FFFFF) | 0x3F800000, jnp.float32)
                for c in R]
        big = [mant[c] > SQRT2 for c in R]
        mant = [jnp.where(big[c], 0.5 * mant[c], mant[c]) for c in R]
        ef = [(ex[c] + jnp.where(big[c], 1, 0)).astype(jnp.float32) for c in R]
        tt = [mant[c] - 1.0 for c in R]
        p = [L2[6] * tt[c] + L2[5] for c in R]
        for coef in (L2[4], L2[3], L2[2], L2[1], L2[0]):
            p = [p[c] * tt[c] + coef for c in R]
        lnz = [(ef[c] + p[c]) * LN2 for c in R]
        val = [dval[c] + (m3[c] - lnz[c]) for c in R]
        for c in R:
            cur[pl.ds(c * 16, 16)] = val[c]

    def body(k, carry):
        t = 3 * k + 2
        diag(t, a2, a1, a0, dva)
        diag(t, b2, b1, b0, dvb)
        diag(t + 1, a0, a2, a1, dva)
        diag(t + 1, b0, b2, b1, dvb)
        diag(t + 2, a1, a0, a2, dva)
        diag(t + 2, b1, b0, b2, dvb)
        return carry

    lax.fori_loop(0, (N + M - 1) // 3, body, 0)
    o_v[...] = a1[pl.ds(128, 16)]
    pltpu.sync_copy(o_v, out_hbm.at[pair])
    o_v[...] = b1[pl.ds(128, 16)]
    pltpu.sync_copy(o_v, out_hbm.at[pair + 1])


def kernel(seq_a, seq_b):
    d = _dist(seq_a, seq_b)
    out = _sc_dp(d)
    return out[:, 0:1]


# speedup vs baseline: 6.0344x; 1.0808x over previous
import functools

import jax
import jax.numpy as jnp
from jax import lax
from jax.experimental import pallas as pl
from jax.experimental.pallas import tpu as pltpu
from jax.experimental.pallas import tpu_sc as plsc

B, N, M, F = 64, 128, 128, 16
INF = 1000000.0
LN2 = 0.6931471805599453
NCHUNK = 9
W = NCHUNK * 16
PADN = 18432
SQRT2 = 1.4142135

L2 = (-1.5931662034414242e-06, 1.4427138430920823, -0.721039027838375,
      0.4793922956076517, -0.36882168336927695, 0.3216487808117249,
      -0.20098562628855351)


def _dist_body(a_ref, b_ref, o_ref):
    a = a_ref[...]
    b = b_ref[...]
    ab = lax.dot_general(a, b, (((2,), (2,)), ((0,), (0,))),
                         preferred_element_type=jnp.float32)
    na = jnp.sum(a * a, axis=-1)
    nb = jnp.sum(b * b, axis=-1)
    d = na[:, :, None] + nb[:, None, :] - 2.0 * ab
    o_ref[:, 128:128 + N * M] = d.reshape(B, N * M)
    o_ref[:, 0:128] = jnp.full((B, 128), INF, jnp.float32)
    o_ref[:, 128 + N * M:] = jnp.full((B, PADN - 128 - N * M), INF,
                                      jnp.float32)


def _dist(a, b):
    return pl.pallas_call(
        _dist_body,
        out_shape=jax.ShapeDtypeStruct((B, PADN), jnp.float32),
    )(a, b)


@functools.partial(
    pl.kernel,
    out_type=jax.ShapeDtypeStruct((B, 16), jnp.float32),
    mesh=plsc.VectorSubcoreMesh(core_axis_name="c", subcore_axis_name="s"),
    compiler_params=pltpu.CompilerParams(needs_layout_passes=False),
    scratch_types=[
        pltpu.VMEM((PADN,), jnp.float32),
        pltpu.VMEM((PADN,), jnp.float32),
        pltpu.VMEM((W,), jnp.float32),
        pltpu.VMEM((W,), jnp.float32),
        pltpu.VMEM((W,), jnp.float32),
        pltpu.VMEM((W,), jnp.float32),
        pltpu.VMEM((W,), jnp.float32),
        pltpu.VMEM((W,), jnp.float32),
        pltpu.VMEM((16,), jnp.float32),
    ],
)
def _sc_dp(d_hbm, out_hbm, dva, dvb, a0, a1, a2, b0, b1, b2, o_v):
    nc = plsc.get_sparse_core_info().num_cores
    wid = lax.axis_index("s") * nc + lax.axis_index("c")
    iota = lax.iota(jnp.int32, 16)
    inf_v = jnp.full((16,), INF, jnp.float32)
    pair = wid * 2
    pltpu.sync_copy(d_hbm.at[pair], dva)
    pltpu.sync_copy(d_hbm.at[pair + 1], dvb)

    for c in range(NCHUNK):
        sl = pl.ds(c * 16, 16)
        v0 = jnp.where(iota == 0, 0.0, INF) if c == 0 else inf_v
        a0[sl] = v0
        b0[sl] = v0
        a1[sl] = inf_v
        b1[sl] = inf_v
        a2[sl] = inf_v
        b2[sl] = inf_v

    def diag(t, cur, p1, p2, dv, R):
        ish0 = jnp.maximum(iota - 1, 0)
        r_up, r_lf, r_dd, dval = [], [], [], []
        for c in R:
            ish = ish0 if c == 0 else iota + (16 * c - 1)
            r_up.append(p1[pl.ds(c * 16, 16)])
            r_lf.append(plsc.load_gather(p1, [ish]))
            r_dd.append(plsc.load_gather(p2, [ish]))
            dval.append(plsc.load_gather(dv, [iota * 127 + (2032 * c + t - 1)]))
        r_up = dict(zip(R, r_up))
        r_lf = dict(zip(R, r_lf))
        r_dd = dict(zip(R, r_dd))
        dval = dict(zip(R, dval))
        m3 = {c: jnp.minimum(jnp.minimum(r_lf[c], r_up[c]), r_dd[c]) for c in R}
        e1 = {c: jnp.exp(m3[c] - r_lf[c]) for c in R}
        e2 = {c: jnp.exp(m3[c] - r_up[c]) for c in R}
        e3 = {c: jnp.exp(m3[c] - r_dd[c]) for c in R}
        z = {c: (e1[c] + e2[c]) + e3[c] for c in R}
        zi = {c: plsc.bitcast(z[c], jnp.int32) for c in R}
        ex = {c: lax.shift_right_logical(zi[c], 23) - 127 for c in R}
        mant = {c: plsc.bitcast((zi[c] & 0x007FFFFF) | 0x3F800000, jnp.float32)
                for c in R}
        big = {c: mant[c] > SQRT2 for c in R}
        mant = {c: jnp.where(big[c], 0.5 * mant[c], mant[c]) for c in R}
        ef = {c: (ex[c] + jnp.where(big[c], 1, 0)).astype(jnp.float32)
              for c in R}
        tt = {c: mant[c] - 1.0 for c in R}
        p = {c: L2[6] * tt[c] + L2[5] for c in R}
        for coef in (L2[4], L2[3], L2[2], L2[1], L2[0]):
            p = {c: p[c] * tt[c] + coef for c in R}
        lnz = {c: (ef[c] + p[c]) * LN2 for c in R}
        val = {c: dval[c] + (m3[c] - lnz[c]) for c in R}
        for c in R:
            cur[pl.ds(c * 16, 16)] = val[c]

    phases = (
        (0, 10, 0, 1), (10, 21, 0, 3), (21, 31, 0, 5), (31, 42, 0, 7),
        (42, 48, 0, 8), (48, 58, 1, 8), (58, 69, 3, 8), (69, 80, 5, 8),
        (80, 85, 7, 8),
    )
    for (k0, k1, cl, ch) in phases:
        def body(k, carry, _r=range(cl, ch + 1)):
            t = 3 * k + 2
            diag(t, a2, a1, a0, dva, _r)
            diag(t, b2, b1, b0, dvb, _r)
            diag(t + 1, a0, a2, a1, dva, _r)
            diag(t + 1, b0, b2, b1, dvb, _r)
            diag(t + 2, a1, a0, a2, dva, _r)
            diag(t + 2, b1, b0, b2, dvb, _r)
            return carry

        lax.fori_loop(k0, k1, body, 0)
    o_v[...] = a1[pl.ds(128, 16)]
    pltpu.sync_copy(o_v, out_hbm.at[pair])
    o_v[...] = b1[pl.ds(128, 16)]
    pltpu.sync_copy(o_v, out_hbm.at[pair + 1])


def kernel(seq_a, seq_b):
    d = _dist(seq_a, seq_b)
    out = _sc_dp(d)
    return out[:, 0:1]


# speedup vs baseline: 6.3029x; 1.0445x over previous
import functools

import jax
import jax.numpy as jnp
from jax import lax
from jax.experimental import pallas as pl
from jax.experimental.pallas import tpu as pltpu
from jax.experimental.pallas import tpu_sc as plsc

B, N, M, F = 64, 128, 128, 16
INF = 1000000.0
LN2 = 0.6931471805599453
NCHUNK = 9
W = NCHUNK * 16
PADN = 18432
SQRT2 = 1.4142135

L2 = (5.7279970403125767e-05, 1.441730616779857, -0.7265749680085072,
      0.5173228495348644, -0.3200435076272173)


def _dist_body(a_ref, b_ref, o_ref):
    a = a_ref[...]
    b = b_ref[...]
    ab = lax.dot_general(a, b, (((2,), (2,)), ((0,), (0,))),
                         preferred_element_type=jnp.float32)
    na = jnp.sum(a * a, axis=-1)
    nb = jnp.sum(b * b, axis=-1)
    d = na[:, :, None] + nb[:, None, :] - 2.0 * ab
    o_ref[:, 128:128 + N * M] = d.reshape(B, N * M)
    o_ref[:, 0:128] = jnp.full((B, 128), INF, jnp.float32)
    o_ref[:, 128 + N * M:] = jnp.full((B, PADN - 128 - N * M), INF,
                                      jnp.float32)


def _dist(a, b):
    return pl.pallas_call(
        _dist_body,
        out_shape=jax.ShapeDtypeStruct((B, PADN), jnp.float32),
    )(a, b)


@functools.partial(
    pl.kernel,
    out_type=jax.ShapeDtypeStruct((B, 16), jnp.float32),
    mesh=plsc.VectorSubcoreMesh(core_axis_name="c", subcore_axis_name="s"),
    compiler_params=pltpu.CompilerParams(needs_layout_passes=False),
    scratch_types=[
        pltpu.VMEM((PADN,), jnp.float32),
        pltpu.VMEM((PADN,), jnp.float32),
        pltpu.VMEM((W,), jnp.float32),
        pltpu.VMEM((W,), jnp.float32),
        pltpu.VMEM((W,), jnp.float32),
        pltpu.VMEM((W,), jnp.float32),
        pltpu.VMEM((W,), jnp.float32),
        pltpu.VMEM((W,), jnp.float32),
        pltpu.VMEM((16,), jnp.float32),
    ],
)
def _sc_dp(d_hbm, out_hbm, dva, dvb, a0, a1, a2, b0, b1, b2, o_v):
    nc = plsc.get_sparse_core_info().num_cores
    wid = lax.axis_index("s") * nc + lax.axis_index("c")
    iota = lax.iota(jnp.int32, 16)
    inf_v = jnp.full((16,), INF, jnp.float32)
    pair = wid * 2
    pltpu.sync_copy(d_hbm.at[pair], dva)
    pltpu.sync_copy(d_hbm.at[pair + 1], dvb)

    for c in range(NCHUNK):
        sl = pl.ds(c * 16, 16)
        v0 = jnp.where(iota == 0, 0.0, INF) if c == 0 else inf_v
        a0[sl] = v0
        b0[sl] = v0
        a1[sl] = inf_v
        b1[sl] = inf_v
        a2[sl] = inf_v
        b2[sl] = inf_v

    def diag(t, cur, p1, p2, dv, R):
        ish0 = jnp.maximum(iota - 1, 0)
        r_up, r_lf, r_dd, dval = [], [], [], []
        for c in R:
            ish = ish0 if c == 0 else iota + (16 * c - 1)
            r_up.append(p1[pl.ds(c * 16, 16)])
            r_lf.append(plsc.load_gather(p1, [ish]))
            r_dd.append(plsc.load_gather(p2, [ish]))
            dval.append(plsc.load_gather(dv, [iota * 127 + (2032 * c + t - 1)]))
        r_up = dict(zip(R, r_up))
        r_lf = dict(zip(R, r_lf))
        r_dd = dict(zip(R, r_dd))
        dval = dict(zip(R, dval))
        m3 = {c: jnp.minimum(jnp.minimum(r_lf[c], r_up[c]), r_dd[c]) for c in R}
        e1 = {c: jnp.exp(m3[c] - r_lf[c]) for c in R}
        e2 = {c: jnp.exp(m3[c] - r_up[c]) for c in R}
        e3 = {c: jnp.exp(m3[c] - r_dd[c]) for c in R}
        z = {c: (e1[c] + e2[c]) + e3[c] for c in R}
        zi = {c: plsc.bitcast(z[c], jnp.int32) for c in R}
        ex = {c: lax.shift_right_logical(zi[c], 23) - 127 for c in R}
        mant = {c: plsc.bitcast((zi[c] & 0x007FFFFF) | 0x3F800000, jnp.float32)
                for c in R}
        big = {c: mant[c] > SQRT2 for c in R}
        mant = {c: jnp.where(big[c], 0.5 * mant[c], mant[c]) for c in R}
        ef = {c: (ex[c] + jnp.where(big[c], 1, 0)).astype(jnp.float32)
              for c in R}
        tt = {c: mant[c] - 1.0 for c in R}
        p = {c: L2[4] * tt[c] + L2[3] for c in R}
        for coef in (L2[2], L2[1], L2[0]):
            p = {c: p[c] * tt[c] + coef for c in R}
        lnz = {c: (ef[c] + p[c]) * LN2 for c in R}
        val = {c: dval[c] + (m3[c] - lnz[c]) for c in R}
        for c in R:
            cur[pl.ds(c * 16, 16)] = val[c]

    phases = (
        (0, 10, 0, 1), (10, 21, 0, 3), (21, 31, 0, 5), (31, 42, 0, 7),
        (42, 48, 0, 8), (48, 58, 1, 8), (58, 69, 3, 8), (69, 80, 5, 8),
        (80, 85, 7, 8),
    )
    for (k0, k1, cl, ch) in phases:
        def body(k, carry, _r=range(cl, ch + 1)):
            t = 3 * k + 2
            diag(t, a2, a1, a0, dva, _r)
            diag(t, b2, b1, b0, dvb, _r)
            diag(t + 1, a0, a2, a1, dva, _r)
            diag(t + 1, b0, b2, b1, dvb, _r)
            diag(t + 2, a1, a0, a2, dva, _r)
            diag(t + 2, b1, b0, b2, dvb, _r)
            return carry

        lax.fori_loop(k0, k1, body, 0)
    o_v[...] = a1[pl.ds(128, 16)]
    pltpu.sync_copy(o_v, out_hbm.at[pair])
    o_v[...] = b1[pl.ds(128, 16)]
    pltpu.sync_copy(o_v, out_hbm.at[pair + 1])


def kernel(seq_a, seq_b):
    d = _dist(seq_a, seq_b)
    out = _sc_dp(d)
    return out[:, 0:1]
